# baseline (speedup 1.0000x reference)
import functools

import jax
import jax.numpy as jnp
from jax import lax
from jax.experimental import pallas as pl
from jax.experimental.pallas import tpu as pltpu
from jax.experimental.pallas import tpu_sc as plsc

N = 100000
E = 6400000
TEMPERATURE = 1.0
INIT_R = 0.9
DECAY_INTERVAL = 10
DECAY_R = 0.1
FINAL_R = 0.5

NPAD = 102400
TC_ROWS = NPAD // 128


def _tc_body(r_ref, logits_ref, noise_ref, attn_out_ref, loss_ref):
    x = logits_ref[...]
    nz = noise_ref[...]
    r = r_ref[0, 0]
    random_noise = jnp.log(nz) - jnp.log(1.0 - nz)
    node_attn = jax.nn.sigmoid((x + random_noise) / TEMPERATURE)
    na16 = jax.lax.bitcast_convert_type(
        node_attn.astype(jnp.bfloat16), jnp.uint16
    ).astype(jnp.uint32)
    lo = na16[: TC_ROWS // 2]
    hi = na16[TC_ROWS // 2 :]
    attn_out_ref[...] = jax.lax.bitcast_convert_type(
        lo | (hi << 16), jnp.int32
    )
    attn = jax.nn.sigmoid(x)
    il = attn * jnp.log(attn / r + 1e-06) + (1.0 - attn) * jnp.log(
        (1.0 - attn) / (1.0 - r + 1e-06) + 1e-06
    )
    row = lax.broadcasted_iota(jnp.int32, (TC_ROWS, 128), 0)
    col = lax.broadcasted_iota(jnp.int32, (TC_ROWS, 128), 1)
    valid = row * 128 + col < N
    il = jnp.where(valid, il, 0.0)
    loss_ref[0, 0] = jnp.sum(il) * (1.0 / N)


def _node_attn_and_loss(logits2d, noise2d, r2d):
    return pl.pallas_call(
        _tc_body,
        out_shape=(
            jax.ShapeDtypeStruct((TC_ROWS // 2, 128), jnp.int32),
            jax.ShapeDtypeStruct((1, 1), jnp.float32),
        ),
        in_specs=[
            pl.BlockSpec(memory_space=pltpu.SMEM),
            pl.BlockSpec(memory_space=pltpu.VMEM),
            pl.BlockSpec(memory_space=pltpu.VMEM),
        ],
        out_specs=(
            pl.BlockSpec(memory_space=pltpu.VMEM),
            pl.BlockSpec(memory_space=pltpu.SMEM),
        ),
    )(r2d, logits2d, noise2d)


NUM_WORKERS = 32
TILES = E // 128
TPC = 100
CHUNK = TPC * 128
N_FULL_CHUNKS = TILES // TPC
MAX_PAIRS = (N_FULL_CHUNKS // NUM_WORKERS + 2) // 2
UNROLL = 8
HALF = NPAD // 2


def _lookup(table_v, idx):
    ge = idx >= HALF
    w = jnp.where(ge, idx - HALF, idx)
    g = plsc.load_gather(table_v, [w])
    sh = jnp.where(ge, 16, 0)
    return plsc.bitcast((g >> sh) << 16, jnp.float32)


def _gather_mul(table_v, ebuf, ov, ntiles, unroll):
    @plsc.parallel_loop(0, ntiles, unroll=unroll)
    def _(i):
        for p in range(8):
            s_idx = ebuf[i, 0, pl.ds(p * 16, 16)]
            d_idx = ebuf[i, 1, pl.ds(p * 16, 16)]
            sa = _lookup(table_v, s_idx)
            da = _lookup(table_v, d_idx)
            ov[pl.ds(i * 128 + p * 16, 16)] = sa * da


def _sc_body(table_hbm, edges_hbm, out_hbm, table_v, e0, e1, o0, o1,
             sin0, sin1, sout0, sout1):
    wid = lax.axis_index("s") * 2 + lax.axis_index("c")
    ebufs = (e0, e1)
    outs = (o0, o1)
    sin = (sin0, sin1)
    sout = (sout0, sout1)
    nc = (N_FULL_CHUNKS - wid + NUM_WORKERS - 1) // NUM_WORKERS

    def start_in(ci, b):
        toff = (wid + ci * NUM_WORKERS) * TPC
        pltpu.make_async_copy(
            edges_hbm.at[pl.ds(toff, TPC)], ebufs[b], sin[b]
        ).start()

    def wait_in(b):
        pltpu.make_async_copy(
            edges_hbm.at[pl.ds(0, TPC)], ebufs[b], sin[b]
        ).wait()

    def start_out(ci, b):
        off = (wid + ci * NUM_WORKERS) * CHUNK
        pltpu.make_async_copy(
            outs[b], out_hbm.at[pl.ds(off, CHUNK)], sout[b]
        ).start()

    def wait_out(b):
        pltpu.make_async_copy(
            outs[b], out_hbm.at[pl.ds(0, CHUNK)], sout[b]
        ).wait()

    def compute(b):
        _gather_mul(table_v, ebufs[b], outs[b], TPC, UNROLL)

    start_in(0, 0)
    pltpu.sync_copy(table_hbm, table_v)

    def pair_body(pi, _):
        c0 = pi * 2
        c1 = c0 + 1

        @pl.when(c1 < nc)
        def _():
            start_in(c1, 1)

        @pl.when(c0 < nc)
        def _():
            wait_in(0)

            @pl.when(pi > 0)
            def _():
                wait_out(0)

            compute(0)
            start_out(c0, 0)

        @pl.when(c0 + 2 < nc)
        def _():
            start_in(c0 + 2, 0)

        @pl.when(c1 < nc)
        def _():
            wait_in(1)

            @pl.when(pi > 0)
            def _():
                wait_out(1)

            compute(1)
            start_out(c1, 1)

        return 0

    lax.fori_loop(0, MAX_PAIRS, pair_body, 0)
    wait_out(0)
    wait_out(1)


_sc_gather = functools.partial(
    pl.kernel,
    out_type=jax.ShapeDtypeStruct((E,), jnp.float32),
    mesh=plsc.VectorSubcoreMesh(core_axis_name="c", subcore_axis_name="s"),
    compiler_params=pltpu.CompilerParams(needs_layout_passes=False),
    scratch_types=[
        pltpu.VMEM((HALF,), jnp.int32),
        pltpu.VMEM((TPC, 2, 128), jnp.int32),
        pltpu.VMEM((TPC, 2, 128), jnp.int32),
        pltpu.VMEM((CHUNK,), jnp.float32),
        pltpu.VMEM((CHUNK,), jnp.float32),
        pltpu.SemaphoreType.DMA,
        pltpu.SemaphoreType.DMA,
        pltpu.SemaphoreType.DMA,
        pltpu.SemaphoreType.DMA,
    ],
)(_sc_body)


def kernel(attn_log_logits, noise, edge_index, epoch):
    r = jnp.maximum(
        INIT_R - (epoch // DECAY_INTERVAL) * DECAY_R, FINAL_R
    ).astype(jnp.float32)
    r2d = r.reshape(1, 1)
    logits_flat = attn_log_logits.reshape(-1)
    noise_flat = noise.reshape(-1)
    logits2d = jnp.pad(logits_flat, (0, NPAD - N)).reshape(TC_ROWS, 128)
    noise2d = jnp.pad(noise_flat, (0, NPAD - N), constant_values=0.5).reshape(
        TC_ROWS, 128
    )
    attn2d, loss11 = _node_attn_and_loss(logits2d, noise2d, r2d)
    table = attn2d.reshape(-1)
    edges3d = edge_index.reshape(2, E // 128, 128).transpose(1, 0, 2)
    edge_attn = _sc_gather(table, edges3d)
    return edge_attn.reshape(E, 1), loss11[0, 0]


# speedup vs baseline: 1.1948x; 1.1948x over previous
import functools

import jax
import jax.numpy as jnp
from jax import lax
from jax.experimental import pallas as pl
from jax.experimental.pallas import tpu as pltpu
from jax.experimental.pallas import tpu_sc as plsc

N = 100000
E = 6400000
TEMPERATURE = 1.0
INIT_R = 0.9
DECAY_INTERVAL = 10
DECAY_R = 0.1
FINAL_R = 0.5

NPAD = 102400
TC_ROWS = NPAD // 128


def _tc_body(r_ref, logits_ref, noise_ref, attn_out_ref, loss_ref):
    x = logits_ref[...]
    nz = noise_ref[...]
    r = r_ref[0, 0]
    random_noise = jnp.log(nz) - jnp.log(1.0 - nz)
    node_attn = jax.nn.sigmoid((x + random_noise) / TEMPERATURE)
    na16 = jax.lax.bitcast_convert_type(
        node_attn.astype(jnp.bfloat16), jnp.uint16
    ).astype(jnp.uint32)
    lo = na16[: TC_ROWS // 2]
    hi = na16[TC_ROWS // 2 :]
    attn_out_ref[...] = jax.lax.bitcast_convert_type(
        lo | (hi << 16), jnp.int32
    )
    attn = jax.nn.sigmoid(x)
    il = attn * jnp.log(attn / r + 1e-06) + (1.0 - attn) * jnp.log(
        (1.0 - attn) / (1.0 - r + 1e-06) + 1e-06
    )
    row = lax.broadcasted_iota(jnp.int32, (TC_ROWS, 128), 0)
    col = lax.broadcasted_iota(jnp.int32, (TC_ROWS, 128), 1)
    valid = row * 128 + col < N
    il = jnp.where(valid, il, 0.0)
    loss_ref[0, 0] = jnp.sum(il) * (1.0 / N)


def _node_attn_and_loss(logits2d, noise2d, r2d):
    return pl.pallas_call(
        _tc_body,
        out_shape=(
            jax.ShapeDtypeStruct((TC_ROWS // 2, 128), jnp.int32),
            jax.ShapeDtypeStruct((1, 1), jnp.float32),
        ),
        in_specs=[
            pl.BlockSpec(memory_space=pltpu.SMEM),
            pl.BlockSpec(memory_space=pltpu.VMEM),
            pl.BlockSpec(memory_space=pltpu.VMEM),
        ],
        out_specs=(
            pl.BlockSpec(memory_space=pltpu.VMEM),
            pl.BlockSpec(memory_space=pltpu.SMEM),
        ),
    )(r2d, logits2d, noise2d)


NUM_WORKERS = 32
TILES = E // 128
TPC = 100
CHUNK = TPC * 128
N_FULL_CHUNKS = TILES // TPC
MAX_PAIRS = (N_FULL_CHUNKS // NUM_WORKERS + 2) // 2
UNROLL = 2
HALF = NPAD // 2


def _lookup(table_v, idx):
    ge = idx >= HALF
    w = jnp.where(ge, idx - HALF, idx)
    g = plsc.load_gather(table_v, [w])
    sh = jnp.where(ge, 16, 0)
    return plsc.bitcast((g >> sh) << 16, jnp.float32)


def _gather_mul(table_v, ebuf, ov, ntiles, unroll):
    @plsc.parallel_loop(0, ntiles, unroll=unroll)
    def _(i):
        for p in range(8):
            s_idx = ebuf[i, 0, pl.ds(p * 16, 16)]
            d_idx = ebuf[i, 1, pl.ds(p * 16, 16)]
            sa = _lookup(table_v, s_idx)
            da = _lookup(table_v, d_idx)
            ov[pl.ds(i * 128 + p * 16, 16)] = sa * da


def _sc_body(table_hbm, edges_hbm, out_hbm, table_v, e0, e1, o0, o1,
             sin0, sin1, sout0, sout1):
    wid = lax.axis_index("s") * 2 + lax.axis_index("c")
    ebufs = (e0, e1)
    outs = (o0, o1)
    sin = (sin0, sin1)
    sout = (sout0, sout1)
    nc = (N_FULL_CHUNKS - wid + NUM_WORKERS - 1) // NUM_WORKERS

    def start_in(ci, b):
        toff = (wid + ci * NUM_WORKERS) * TPC
        pltpu.make_async_copy(
            edges_hbm.at[pl.ds(toff, TPC)], ebufs[b], sin[b]
        ).start()

    def wait_in(b):
        pltpu.make_async_copy(
            edges_hbm.at[pl.ds(0, TPC)], ebufs[b], sin[b]
        ).wait()

    def start_out(ci, b):
        off = (wid + ci * NUM_WORKERS) * CHUNK
        pltpu.make_async_copy(
            outs[b], out_hbm.at[pl.ds(off, CHUNK)], sout[b]
        ).start()

    def wait_out(b):
        pltpu.make_async_copy(
            outs[b], out_hbm.at[pl.ds(0, CHUNK)], sout[b]
        ).wait()

    def compute(b):
        _gather_mul(table_v, ebufs[b], outs[b], TPC, UNROLL)

    start_in(0, 0)
    pltpu.sync_copy(table_hbm, table_v)

    def pair_body(pi, _):
        c0 = pi * 2
        c1 = c0 + 1

        @pl.when(c1 < nc)
        def _():
            start_in(c1, 1)

        @pl.when(c0 < nc)
        def _():
            wait_in(0)

            @pl.when(pi > 0)
            def _():
                wait_out(0)

            compute(0)
            start_out(c0, 0)

        @pl.when(c0 + 2 < nc)
        def _():
            start_in(c0 + 2, 0)

        @pl.when(c1 < nc)
        def _():
            wait_in(1)

            @pl.when(pi > 0)
            def _():
                wait_out(1)

            compute(1)
            start_out(c1, 1)

        return 0

    lax.fori_loop(0, MAX_PAIRS, pair_body, 0)
    wait_out(0)
    wait_out(1)


_sc_gather = functools.partial(
    pl.kernel,
    out_type=jax.ShapeDtypeStruct((E,), jnp.float32),
    mesh=plsc.VectorSubcoreMesh(core_axis_name="c", subcore_axis_name="s"),
    compiler_params=pltpu.CompilerParams(needs_layout_passes=False),
    scratch_types=[
        pltpu.VMEM((HALF,), jnp.int32),
        pltpu.VMEM((TPC, 2, 128), jnp.int32),
        pltpu.VMEM((TPC, 2, 128), jnp.int32),
        pltpu.VMEM((CHUNK,), jnp.float32),
        pltpu.VMEM((CHUNK,), jnp.float32),
        pltpu.SemaphoreType.DMA,
        pltpu.SemaphoreType.DMA,
        pltpu.SemaphoreType.DMA,
        pltpu.SemaphoreType.DMA,
    ],
)(_sc_body)


def kernel(attn_log_logits, noise, edge_index, epoch):
    r = jnp.maximum(
        INIT_R - (epoch // DECAY_INTERVAL) * DECAY_R, FINAL_R
    ).astype(jnp.float32)
    r2d = r.reshape(1, 1)
    logits_flat = attn_log_logits.reshape(-1)
    noise_flat = noise.reshape(-1)
    logits2d = jnp.pad(logits_flat, (0, NPAD - N)).reshape(TC_ROWS, 128)
    noise2d = jnp.pad(noise_flat, (0, NPAD - N), constant_values=0.5).reshape(
        TC_ROWS, 128
    )
    attn2d, loss11 = _node_attn_and_loss(logits2d, noise2d, r2d)
    table = attn2d.reshape(-1)
    edges3d = edge_index.reshape(2, E // 128, 128).transpose(1, 0, 2)
    edge_attn = _sc_gather(table, edges3d)
    return edge_attn.reshape(E, 1), loss11[0, 0]


# speedup vs baseline: 1.2024x; 1.0064x over previous
import functools

import jax
import jax.numpy as jnp
from jax import lax
from jax.experimental import pallas as pl
from jax.experimental.pallas import tpu as pltpu
from jax.experimental.pallas import tpu_sc as plsc

N = 100000
E = 6400000
TEMPERATURE = 1.0
INIT_R = 0.9
DECAY_INTERVAL = 10
DECAY_R = 0.1
FINAL_R = 0.5

NPAD = 102400
TC_ROWS = NPAD // 128


def _tc_body(r_ref, logits_ref, noise_ref, attn_out_ref, loss_ref):
    x = logits_ref[...]
    nz = noise_ref[...]
    r = r_ref[0, 0]
    random_noise = jnp.log(nz) - jnp.log(1.0 - nz)
    node_attn = jax.nn.sigmoid((x + random_noise) / TEMPERATURE)
    na16 = jax.lax.bitcast_convert_type(
        node_attn.astype(jnp.bfloat16), jnp.uint16
    ).astype(jnp.uint32)
    lo = na16[: TC_ROWS // 2]
    hi = na16[TC_ROWS // 2 :]
    attn_out_ref[...] = jax.lax.bitcast_convert_type(
        lo | (hi << 16), jnp.int32
    )
    attn = jax.nn.sigmoid(x)
    il = attn * jnp.log(attn / r + 1e-06) + (1.0 - attn) * jnp.log(
        (1.0 - attn) / (1.0 - r + 1e-06) + 1e-06
    )
    row = lax.broadcasted_iota(jnp.int32, (TC_ROWS, 128), 0)
    col = lax.broadcasted_iota(jnp.int32, (TC_ROWS, 128), 1)
    valid = row * 128 + col < N
    il = jnp.where(valid, il, 0.0)
    loss_ref[0, 0] = jnp.sum(il) * (1.0 / N)


def _node_attn_and_loss(logits2d, noise2d, r2d):
    return pl.pallas_call(
        _tc_body,
        out_shape=(
            jax.ShapeDtypeStruct((TC_ROWS // 2, 128), jnp.int32),
            jax.ShapeDtypeStruct((1, 1), jnp.float32),
        ),
        in_specs=[
            pl.BlockSpec(memory_space=pltpu.SMEM),
            pl.BlockSpec(memory_space=pltpu.VMEM),
            pl.BlockSpec(memory_space=pltpu.VMEM),
        ],
        out_specs=(
            pl.BlockSpec(memory_space=pltpu.VMEM),
            pl.BlockSpec(memory_space=pltpu.SMEM),
        ),
    )(r2d, logits2d, noise2d)


NUM_WORKERS = 32
TILES = E // 128
TPC = 100
CHUNK = TPC * 128
N_FULL_CHUNKS = TILES // TPC
MAX_PAIRS = (N_FULL_CHUNKS // NUM_WORKERS + 2) // 2
UNROLL = 4
HALF = NPAD // 2


def _lookup(table_v, idx):
    ge = idx >= HALF
    w = jnp.where(ge, idx - HALF, idx)
    g = plsc.load_gather(table_v, [w])
    sh = jnp.where(ge, 16, 0)
    return plsc.bitcast((g >> sh) << 16, jnp.float32)


def _gather_mul(table_v, ebuf, ov, ntiles, unroll):
    @plsc.parallel_loop(0, ntiles, unroll=unroll)
    def _(i):
        for p in range(8):
            s_idx = ebuf[i, 0, pl.ds(p * 16, 16)]
            d_idx = ebuf[i, 1, pl.ds(p * 16, 16)]
            sa = _lookup(table_v, s_idx)
            da = _lookup(table_v, d_idx)
            ov[pl.ds(i * 128 + p * 16, 16)] = sa * da


def _sc_body(table_hbm, edges_hbm, out_hbm, table_v, e0, e1, o0, o1,
             sin0, sin1, sout0, sout1):
    wid = lax.axis_index("s") * 2 + lax.axis_index("c")
    ebufs = (e0, e1)
    outs = (o0, o1)
    sin = (sin0, sin1)
    sout = (sout0, sout1)
    nc = (N_FULL_CHUNKS - wid + NUM_WORKERS - 1) // NUM_WORKERS

    def start_in(ci, b):
        toff = (wid + ci * NUM_WORKERS) * TPC
        pltpu.make_async_copy(
            edges_hbm.at[pl.ds(toff, TPC)], ebufs[b], sin[b]
        ).start()

    def wait_in(b):
        pltpu.make_async_copy(
            edges_hbm.at[pl.ds(0, TPC)], ebufs[b], sin[b]
        ).wait()

    def start_out(ci, b):
        off = (wid + ci * NUM_WORKERS) * CHUNK
        pltpu.make_async_copy(
            outs[b], out_hbm.at[pl.ds(off, CHUNK)], sout[b]
        ).start()

    def wait_out(b):
        pltpu.make_async_copy(
            outs[b], out_hbm.at[pl.ds(0, CHUNK)], sout[b]
        ).wait()

    def compute(b):
        _gather_mul(table_v, ebufs[b], outs[b], TPC, UNROLL)

    start_in(0, 0)
    pltpu.sync_copy(table_hbm, table_v)

    def pair_body(pi, _):
        c0 = pi * 2
        c1 = c0 + 1

        @pl.when(c1 < nc)
        def _():
            start_in(c1, 1)

        @pl.when(c0 < nc)
        def _():
            wait_in(0)

            @pl.when(pi > 0)
            def _():
                wait_out(0)

            compute(0)
            start_out(c0, 0)

        @pl.when(c0 + 2 < nc)
        def _():
            start_in(c0 + 2, 0)

        @pl.when(c1 < nc)
        def _():
            wait_in(1)

            @pl.when(pi > 0)
            def _():
                wait_out(1)

            compute(1)
            start_out(c1, 1)

        return 0

    lax.fori_loop(0, MAX_PAIRS, pair_body, 0)
    wait_out(0)
    wait_out(1)


_sc_gather = functools.partial(
    pl.kernel,
    out_type=jax.ShapeDtypeStruct((E,), jnp.float32),
    mesh=plsc.VectorSubcoreMesh(core_axis_name="c", subcore_axis_name="s"),
    compiler_params=pltpu.CompilerParams(needs_layout_passes=False),
    scratch_types=[
        pltpu.VMEM((HALF,), jnp.int32),
        pltpu.VMEM((TPC, 2, 128), jnp.int32),
        pltpu.VMEM((TPC, 2, 128), jnp.int32),
        pltpu.VMEM((CHUNK,), jnp.float32),
        pltpu.VMEM((CHUNK,), jnp.float32),
        pltpu.SemaphoreType.DMA,
        pltpu.SemaphoreType.DMA,
        pltpu.SemaphoreType.DMA,
        pltpu.SemaphoreType.DMA,
    ],
)(_sc_body)


def kernel(attn_log_logits, noise, edge_index, epoch):
    r = jnp.maximum(
        INIT_R - (epoch // DECAY_INTERVAL) * DECAY_R, FINAL_R
    ).astype(jnp.float32)
    r2d = r.reshape(1, 1)
    logits_flat = attn_log_logits.reshape(-1)
    noise_flat = noise.reshape(-1)
    logits2d = jnp.pad(logits_flat, (0, NPAD - N)).reshape(TC_ROWS, 128)
    noise2d = jnp.pad(noise_flat, (0, NPAD - N), constant_values=0.5).reshape(
        TC_ROWS, 128
    )
    attn2d, loss11 = _node_attn_and_loss(logits2d, noise2d, r2d)
    table = attn2d.reshape(-1)
    edges3d = edge_index.reshape(2, E // 128, 128).transpose(1, 0, 2)
    edge_attn = _sc_gather(table, edges3d)
    return edge_attn.reshape(E, 1), loss11[0, 0]
